# Initial kernel scaffold; baseline (speedup 1.0000x reference)
import jax
import jax.numpy as jnp
from jax.experimental import pallas as pl


def kernel(emb, node_emb_encoded, teacher_forcing, W_in1, b_in1, W_in2, b_in2, W_ih0, W_hh0, b_ih0, b_hh0, W_ih1, W_hh1, b_ih1, b_hh1, W_out, b_out):
    raise NotImplementedError("write your pallas kernel here")


# speedup vs baseline: 6.0468x; 6.0468x over previous
import functools

import jax
import jax.numpy as jnp
from jax import lax
from jax.experimental import pallas as pl
from jax.experimental.pallas import tpu as pltpu
from jax.experimental.pallas import tpu_sc as plsc

EMB_DIM = 256
NODE_DIM = 128
HIDDEN = 512
NUM_NODES = 64
BATCH = 128
STEPS = NUM_NODES

SC_CORES = 2
SC_SUBCORES = 16
SC_WORKERS = SC_CORES * SC_SUBCORES
SAMPLES_PER_WORKER = BATCH // SC_WORKERS
LANES = 16
NCHUNK = NUM_NODES // LANES


def _mm(a, b):
    return jax.lax.dot_general(
        a, b, (((1,), (0,)), ((), ())), preferred_element_type=jnp.float32
    )


def _decode_body(emb_ref, a_in1_ref, b_in1_ref, a_in2_ref, b_in2_ref,
                 a_cat0_ref, bias0_ref, a_cat1_ref, bias1_ref,
                 a_out_ref, b_out_ref, preds_ref):
    h = jax.nn.relu(_mm(emb_ref[...], a_in1_ref[...]) + b_in1_ref[...])
    hx = _mm(h, a_in2_ref[...]) + b_in2_ref[...]
    h0 = hx[:, :HIDDEN]
    h1 = hx[:, HIDDEN:]
    c0 = jnp.zeros((BATCH, HIDDEN), jnp.float32)
    c1 = jnp.zeros((BATCH, HIDDEN), jnp.float32)
    x = jnp.zeros((BATCH, NODE_DIM), jnp.float32)

    a_cat0 = a_cat0_ref[...]
    bias0 = bias0_ref[...]
    a_cat1 = a_cat1_ref[...]
    bias1 = bias1_ref[...]
    a_out = a_out_ref[...]
    b_out = b_out_ref[...]

    def cell(xh, c, a_cat, bias):
        g = _mm(xh, a_cat) + bias
        i = jax.nn.sigmoid(g[:, 0 * HIDDEN:1 * HIDDEN])
        f = jax.nn.sigmoid(g[:, 1 * HIDDEN:2 * HIDDEN])
        gg = jnp.tanh(g[:, 2 * HIDDEN:3 * HIDDEN])
        o = jax.nn.sigmoid(g[:, 3 * HIDDEN:4 * HIDDEN])
        c_new = f * c + i * gg
        h_new = o * jnp.tanh(c_new)
        return h_new, c_new

    def step(t, carry):
        x, h0, c0, h1, c1 = carry
        h0, c0 = cell(jnp.concatenate([x, h0], axis=1), c0, a_cat0, bias0)
        h1, c1 = cell(jnp.concatenate([h0, h1], axis=1), c1, a_cat1, bias1)
        x = _mm(h1, a_out) + b_out
        preds_ref[t] = x
        return (x, h0, c0, h1, c1)

    lax.fori_loop(0, STEPS, step, (x, h0, c0, h1, c1))


def _dist_body(preds_ref, enc_ref, d_ref):
    bg = enc_ref.shape[0]
    for j in range(bg):
        p = preds_ref[:, j, :]
        e = enc_ref[j]
        cross = jax.lax.dot_general(
            p, e, (((1,), (1,)), ((), ())),
            preferred_element_type=jnp.float32)
        esq = jnp.sum(e * e, axis=1)
        d_ref[j] = esq[None, :] - 2.0 * cross


def _sc_select_body(d_hbm, preds_hbm, out_hbm, d_v, idx_v, rows_v,
                    red_f, red_i, sem):
    wid = lax.axis_index("s") * SC_CORES + lax.axis_index("c")
    iota = lax.iota(jnp.int32, LANES)
    big = jnp.float32(1e30)

    def lane_min_f(v):
        for sh in (8, 4, 2, 1):
            red_f[...] = v
            v = jnp.minimum(v, plsc.load_gather(red_f, [iota ^ sh]))
        return v

    def lane_min_i(v):
        for sh in (8, 4, 2, 1):
            red_i[...] = v
            v = jnp.minimum(v, plsc.load_gather(red_i, [iota ^ sh]))
        return v

    for j in range(SAMPLES_PER_WORKER):
        b = wid * SAMPLES_PER_WORKER + j
        pltpu.sync_copy(d_hbm.at[pl.ds(b * STEPS * NUM_NODES,
                                       STEPS * NUM_NODES)], d_v)

        def gstep(t, carry):
            pen = carry[:NCHUNK]
            inv = carry[NCHUNK:]
            m = [d_v[pl.ds(t * NUM_NODES + cc * LANES, LANES)] + pen[cc]
                 for cc in range(NCHUNK)]
            mm = jnp.minimum(jnp.minimum(m[0], m[1]),
                             jnp.minimum(m[2], m[3]))
            gmin = lane_min_f(mm)
            cand = [jnp.where(m[cc] == gmin, iota + cc * LANES,
                              jnp.int32(NUM_NODES))
                    for cc in range(NCHUNK)]
            ci = jnp.minimum(jnp.minimum(cand[0], cand[1]),
                             jnp.minimum(cand[2], cand[3]))
            idx = lane_min_i(ci)
            hit = [iota + cc * LANES == idx for cc in range(NCHUNK)]
            pen = [jnp.where(hit[cc], big, pen[cc]) for cc in range(NCHUNK)]
            inv = [jnp.where(hit[cc], t, inv[cc]) for cc in range(NCHUNK)]
            return tuple(pen) + tuple(inv)

        zf = jnp.zeros((LANES,), jnp.float32)
        zi = jnp.zeros((LANES,), jnp.int32)
        carry = lax.fori_loop(0, STEPS, gstep, (zf,) * NCHUNK + (zi,) * NCHUNK)
        inv = carry[NCHUNK:]
        for cc in range(NCHUNK):
            idx_v[pl.ds(cc * LANES, LANES)] = inv[cc] * BATCH + b
        pltpu.async_copy(preds_hbm.at[idx_v], rows_v, sem).wait()
        pltpu.sync_copy(rows_v, out_hbm.at[pl.ds(b * NUM_NODES, NUM_NODES)])


@functools.cache
def _sc_select():
    return pl.kernel(
        _sc_select_body,
        out_type=jax.ShapeDtypeStruct((BATCH * NUM_NODES, NODE_DIM),
                                      jnp.float32),
        mesh=plsc.VectorSubcoreMesh(core_axis_name="c", subcore_axis_name="s"),
        scratch_types=[
            pltpu.VMEM((STEPS * NUM_NODES,), jnp.float32),
            pltpu.VMEM((NUM_NODES,), jnp.int32),
            pltpu.VMEM((NUM_NODES, NODE_DIM), jnp.float32),
            pltpu.VMEM((LANES,), jnp.float32),
            pltpu.VMEM((LANES,), jnp.int32),
            pltpu.SemaphoreType.DMA,
        ],
        compiler_params=pltpu.CompilerParams(needs_layout_passes=False),
    )


def kernel(emb, node_emb_encoded, teacher_forcing, W_in1, b_in1, W_in2, b_in2,
           W_ih0, W_hh0, b_ih0, b_hh0, W_ih1, W_hh1, b_ih1, b_hh1,
           W_out, b_out):
    del teacher_forcing
    a_in1 = W_in1.T
    a_in2 = W_in2.T
    a_cat0 = jnp.concatenate([W_ih0.T, W_hh0.T], axis=0)
    a_cat1 = jnp.concatenate([W_ih1.T, W_hh1.T], axis=0)
    bias0 = (b_ih0 + b_hh0)[None, :]
    bias1 = (b_ih1 + b_hh1)[None, :]
    a_out = W_out.T

    preds = pl.pallas_call(
        _decode_body,
        out_shape=jax.ShapeDtypeStruct((STEPS, BATCH, NODE_DIM), jnp.float32),
    )(emb, a_in1, b_in1[None, :], a_in2, b_in2[None, :],
      a_cat0, bias0, a_cat1, bias1, a_out, b_out[None, :])

    BG = 16
    d = pl.pallas_call(
        _dist_body,
        grid=(BATCH // BG,),
        in_specs=[
            pl.BlockSpec((STEPS, BG, NODE_DIM), lambda i: (0, i, 0)),
            pl.BlockSpec((BG, NUM_NODES, NODE_DIM), lambda i: (i, 0, 0)),
        ],
        out_specs=pl.BlockSpec((BG, STEPS, NUM_NODES), lambda i: (i, 0, 0)),
        out_shape=jax.ShapeDtypeStruct((BATCH, STEPS, NUM_NODES), jnp.float32),
    )(preds, node_emb_encoded)

    out_flat = _sc_select()(d.reshape(-1), preds.reshape(-1, NODE_DIM))
    return out_flat.reshape(BATCH, NUM_NODES, NODE_DIM)


# speedup vs baseline: 10.5573x; 1.7459x over previous
import functools

import jax
import jax.numpy as jnp
from jax import lax
from jax.experimental import pallas as pl
from jax.experimental.pallas import tpu as pltpu
from jax.experimental.pallas import tpu_sc as plsc

EMB_DIM = 256
NODE_DIM = 128
HIDDEN = 512
NUM_NODES = 64
BATCH = 128
STEPS = NUM_NODES

SC_CORES = 2
SC_SUBCORES = 16
SC_WORKERS = SC_CORES * SC_SUBCORES
SAMPLES_PER_WORKER = BATCH // SC_WORKERS
LANES = 16
NCHUNK = NUM_NODES // LANES


def _mm(a, b):
    return jax.lax.dot_general(
        a, b, (((1,), (0,)), ((), ())), preferred_element_type=jnp.float32
    )


def _decode_body(emb_ref, a_in1_ref, b_in1_ref, a_in2_ref, b_in2_ref,
                 a_cat0_ref, bias0_ref, a_cat1_ref, bias1_ref,
                 a_out_ref, b_out_ref, enc_ref, preds_ref, d_ref):
    h = jax.nn.relu(_mm(emb_ref[...], a_in1_ref[...]) + b_in1_ref[...])
    hx = _mm(h, a_in2_ref[...]) + b_in2_ref[...]
    h0 = hx[:, :HIDDEN]
    h1 = hx[:, HIDDEN:]
    c0 = jnp.zeros((BATCH, HIDDEN), jnp.float32)
    c1 = jnp.zeros((BATCH, HIDDEN), jnp.float32)
    x = jnp.zeros((BATCH, NODE_DIM), jnp.float32)

    a_cat0 = a_cat0_ref[...]
    bias0 = bias0_ref[...]
    a_cat1 = a_cat1_ref[...]
    bias1 = bias1_ref[...]
    a_out = a_out_ref[...]
    b_out = b_out_ref[...]

    def cell(xh, c, a_cat, bias):
        g = _mm(xh, a_cat) + bias
        i = jax.nn.sigmoid(g[:, 0 * HIDDEN:1 * HIDDEN])
        f = jax.nn.sigmoid(g[:, 1 * HIDDEN:2 * HIDDEN])
        gg = jnp.tanh(g[:, 2 * HIDDEN:3 * HIDDEN])
        o = jax.nn.sigmoid(g[:, 3 * HIDDEN:4 * HIDDEN])
        c_new = f * c + i * gg
        h_new = o * jnp.tanh(c_new)
        return h_new, c_new

    def step(t, carry):
        x, h0, c0, h1, c1 = carry
        h0, c0 = cell(jnp.concatenate([x, h0], axis=1), c0, a_cat0, bias0)
        h1, c1 = cell(jnp.concatenate([h0, h1], axis=1), c1, a_cat1, bias1)
        x = _mm(h1, a_out) + b_out
        preds_ref[t] = x
        return (x, h0, c0, h1, c1)

    lax.fori_loop(0, STEPS, step, (x, h0, c0, h1, c1))

    p_all = preds_ref[...] * -2.0
    for j in range(BATCH):
        p = p_all[:, j, :]
        e = enc_ref[j]
        d_ref[j, 1:] = jax.lax.dot_general(
            p, e, (((1,), (1,)), ((), ())),
            preferred_element_type=jnp.float32)
        d_ref[j, 0] = jnp.sum(e * e, axis=1)


BLK = (STEPS + 1) * NUM_NODES


def _sc_select_body(d_hbm, preds_hbm, out_hbm, d_v0, d_v1, idx_v,
                    rows_v, red_f, red_i, sem0, sem1, gsem):
    wid = lax.axis_index("s") * SC_CORES + lax.axis_index("c")
    iota = lax.iota(jnp.int32, LANES)
    big = jnp.float32(1e30)

    def lane_min_f(v):
        for sh in (8, 4, 2, 1):
            red_f[...] = v
            v = jnp.minimum(v, plsc.load_gather(red_f, [iota ^ sh]))
        return v

    def lane_min_i(v):
        for sh in (8, 4, 2, 1):
            red_i[...] = v
            v = jnp.minimum(v, plsc.load_gather(red_i, [iota ^ sh]))
        return v

    b0 = wid * SAMPLES_PER_WORKER
    bufs = (d_v0, d_v1)
    sems = (sem0, sem1)
    copies = {0: pltpu.async_copy(d_hbm.at[pl.ds(b0 * BLK, BLK)], d_v0, sem0)}
    for j in range(SAMPLES_PER_WORKER):
        b = b0 + j
        copies[j].wait()
        if j + 1 < SAMPLES_PER_WORKER:
            copies[j + 1] = pltpu.async_copy(
                d_hbm.at[pl.ds((b + 1) * BLK, BLK)],
                bufs[(j + 1) % 2], sems[(j + 1) % 2])
        d_v = bufs[j % 2]

        def gstep(t, carry):
            pen = carry[:NCHUNK]
            inv = carry[NCHUNK:]
            m = [d_v[pl.ds((t + 1) * NUM_NODES + cc * LANES, LANES)] + pen[cc]
                 for cc in range(NCHUNK)]
            mm = jnp.minimum(jnp.minimum(m[0], m[1]),
                             jnp.minimum(m[2], m[3]))
            gmin = lane_min_f(mm)
            cand = [jnp.where(m[cc] == gmin, iota + cc * LANES,
                              jnp.int32(NUM_NODES))
                    for cc in range(NCHUNK)]
            ci = jnp.minimum(jnp.minimum(cand[0], cand[1]),
                             jnp.minimum(cand[2], cand[3]))
            idx = lane_min_i(ci)
            hit = [iota + cc * LANES == idx for cc in range(NCHUNK)]
            pen = [jnp.where(hit[cc], big, pen[cc]) for cc in range(NCHUNK)]
            inv = [jnp.where(hit[cc], t, inv[cc]) for cc in range(NCHUNK)]
            return tuple(pen) + tuple(inv)

        pen0 = tuple(d_v[pl.ds(cc * LANES, LANES)] for cc in range(NCHUNK))
        zi = jnp.zeros((LANES,), jnp.int32)
        carry = lax.fori_loop(0, STEPS, gstep, pen0 + (zi,) * NCHUNK)
        inv = carry[NCHUNK:]
        for cc in range(NCHUNK):
            idx_v[pl.ds(cc * LANES, LANES)] = inv[cc] * BATCH + b
        pltpu.async_copy(preds_hbm.at[idx_v], rows_v, gsem).wait()
        pltpu.sync_copy(rows_v, out_hbm.at[pl.ds(b * NUM_NODES, NUM_NODES)])


@functools.cache
def _sc_select():
    return pl.kernel(
        _sc_select_body,
        out_type=jax.ShapeDtypeStruct((BATCH * NUM_NODES, NODE_DIM),
                                      jnp.float32),
        mesh=plsc.VectorSubcoreMesh(core_axis_name="c", subcore_axis_name="s"),
        scratch_types=[
            pltpu.VMEM((BLK,), jnp.float32),
            pltpu.VMEM((BLK,), jnp.float32),
            pltpu.VMEM((NUM_NODES,), jnp.int32),
            pltpu.VMEM((NUM_NODES, NODE_DIM), jnp.float32),
            pltpu.VMEM((LANES,), jnp.float32),
            pltpu.VMEM((LANES,), jnp.int32),
            pltpu.SemaphoreType.DMA,
            pltpu.SemaphoreType.DMA,
            pltpu.SemaphoreType.DMA,
        ],
        compiler_params=pltpu.CompilerParams(needs_layout_passes=False),
    )


def kernel(emb, node_emb_encoded, teacher_forcing, W_in1, b_in1, W_in2, b_in2,
           W_ih0, W_hh0, b_ih0, b_hh0, W_ih1, W_hh1, b_ih1, b_hh1,
           W_out, b_out):
    del teacher_forcing
    a_in1 = W_in1.T
    a_in2 = W_in2.T
    a_cat0 = jnp.concatenate([W_ih0.T, W_hh0.T], axis=0)
    a_cat1 = jnp.concatenate([W_ih1.T, W_hh1.T], axis=0)
    bias0 = (b_ih0 + b_hh0)[None, :]
    bias1 = (b_ih1 + b_hh1)[None, :]
    a_out = W_out.T

    preds, d = pl.pallas_call(
        _decode_body,
        out_shape=[
            jax.ShapeDtypeStruct((STEPS, BATCH, NODE_DIM), jnp.float32),
            jax.ShapeDtypeStruct((BATCH, STEPS + 1, NUM_NODES), jnp.float32),
        ],
    )(emb, a_in1, b_in1[None, :], a_in2, b_in2[None, :],
      a_cat0, bias0, a_cat1, bias1, a_out, b_out[None, :], node_emb_encoded)

    out_flat = _sc_select()(d.reshape(-1), preds.reshape(-1, NODE_DIM))
    return out_flat.reshape(BATCH, NUM_NODES, NODE_DIM)


# speedup vs baseline: 10.9206x; 1.0344x over previous
import functools

import jax
import jax.numpy as jnp
from jax import lax
from jax.experimental import pallas as pl
from jax.experimental.pallas import tpu as pltpu
from jax.experimental.pallas import tpu_sc as plsc

EMB_DIM = 256
NODE_DIM = 128
HIDDEN = 512
NUM_NODES = 64
BATCH = 128
STEPS = NUM_NODES

SC_CORES = 2
SC_SUBCORES = 16
SC_WORKERS = SC_CORES * SC_SUBCORES
SAMPLES_PER_WORKER = BATCH // SC_WORKERS
LANES = 16
NCHUNK = NUM_NODES // LANES


def _mm(a, b):
    return jax.lax.dot_general(
        a, b, (((1,), (0,)), ((), ())), preferred_element_type=jnp.float32
    )


def _decode_body(emb_ref, a_in1_ref, b_in1_ref, a_in2_ref, b_in2_ref,
                 a_cat0_ref, bias0_ref, a_cat1_ref, bias1_ref,
                 a_out_ref, b_out_ref, enc_ref, preds_ref, d_ref):
    h = jax.nn.relu(_mm(emb_ref[...], a_in1_ref[...]) + b_in1_ref[...])
    hx = _mm(h, a_in2_ref[...]) + b_in2_ref[...]
    h0 = hx[:, :HIDDEN]
    h1 = hx[:, HIDDEN:]
    c0 = jnp.zeros((BATCH, HIDDEN), jnp.float32)
    c1 = jnp.zeros((BATCH, HIDDEN), jnp.float32)
    x = jnp.zeros((BATCH, NODE_DIM), jnp.float32)

    a_cat0 = a_cat0_ref[...]
    bias0 = bias0_ref[...]
    a_cat1 = a_cat1_ref[...]
    bias1 = bias1_ref[...]
    a_out = a_out_ref[...]
    b_out = b_out_ref[...]

    def cell(xh, c, a_cat, bias):
        g = _mm(xh, a_cat) + bias
        i = jax.nn.sigmoid(g[:, 0 * HIDDEN:1 * HIDDEN])
        f = jax.nn.sigmoid(g[:, 1 * HIDDEN:2 * HIDDEN])
        gg = jnp.tanh(g[:, 2 * HIDDEN:3 * HIDDEN])
        o = jax.nn.sigmoid(g[:, 3 * HIDDEN:4 * HIDDEN])
        c_new = f * c + i * gg
        h_new = o * jnp.tanh(c_new)
        return h_new, c_new

    def step(t, carry):
        x, h0, c0, h1, c1 = carry
        h0, c0 = cell(jnp.concatenate([x, h0], axis=1), c0, a_cat0, bias0)
        h1, c1 = cell(jnp.concatenate([h0, h1], axis=1), c1, a_cat1, bias1)
        x = _mm(h1, a_out) + b_out
        preds_ref[t] = x
        return (x, h0, c0, h1, c1)

    def step2(u, carry):
        return step(2 * u + 1, step(2 * u, carry))

    lax.fori_loop(0, STEPS // 2, step2, (x, h0, c0, h1, c1))

    p_all = preds_ref[...] * -2.0
    for j in range(BATCH):
        p = p_all[:, j, :]
        e = enc_ref[j]
        d_ref[j, 1:] = jax.lax.dot_general(
            p, e, (((1,), (1,)), ((), ())),
            preferred_element_type=jnp.float32)
        d_ref[j, 0] = jnp.sum(e * e, axis=1)


BLK = (STEPS + 1) * NUM_NODES


def _sc_select_body(d_hbm, preds_hbm, out_hbm, d_v0, d_v1, idx_v,
                    rows_v, red_f, red_i, sem0, sem1, gsem):
    wid = lax.axis_index("s") * SC_CORES + lax.axis_index("c")
    iota = lax.iota(jnp.int32, LANES)
    big = jnp.float32(1e30)

    def lane_min_f(v):
        for sh in (8, 4, 2, 1):
            red_f[...] = v
            v = jnp.minimum(v, plsc.load_gather(red_f, [iota ^ sh]))
        return v

    def lane_min_i(v):
        for sh in (8, 4, 2, 1):
            red_i[...] = v
            v = jnp.minimum(v, plsc.load_gather(red_i, [iota ^ sh]))
        return v

    b0 = wid * SAMPLES_PER_WORKER
    bufs = (d_v0, d_v1)
    sems = (sem0, sem1)
    copies = {0: pltpu.async_copy(d_hbm.at[pl.ds(b0 * BLK, BLK)], d_v0, sem0)}
    for j in range(SAMPLES_PER_WORKER):
        b = b0 + j
        copies[j].wait()
        if j + 1 < SAMPLES_PER_WORKER:
            copies[j + 1] = pltpu.async_copy(
                d_hbm.at[pl.ds((b + 1) * BLK, BLK)],
                bufs[(j + 1) % 2], sems[(j + 1) % 2])
        d_v = bufs[j % 2]

        def gstep(t, carry):
            pen = carry[:NCHUNK]
            inv = carry[NCHUNK:]
            m = [d_v[pl.ds((t + 1) * NUM_NODES + cc * LANES, LANES)] + pen[cc]
                 for cc in range(NCHUNK)]
            mm = jnp.minimum(jnp.minimum(m[0], m[1]),
                             jnp.minimum(m[2], m[3]))
            gmin = lane_min_f(mm)
            cand = [jnp.where(m[cc] == gmin, iota + cc * LANES,
                              jnp.int32(NUM_NODES))
                    for cc in range(NCHUNK)]
            ci = jnp.minimum(jnp.minimum(cand[0], cand[1]),
                             jnp.minimum(cand[2], cand[3]))
            idx = lane_min_i(ci)
            hit = [iota + cc * LANES == idx for cc in range(NCHUNK)]
            pen = [jnp.where(hit[cc], big, pen[cc]) for cc in range(NCHUNK)]
            inv = [jnp.where(hit[cc], t, inv[cc]) for cc in range(NCHUNK)]
            return tuple(pen) + tuple(inv)

        pen0 = tuple(d_v[pl.ds(cc * LANES, LANES)] for cc in range(NCHUNK))
        zi = jnp.zeros((LANES,), jnp.int32)
        carry = lax.fori_loop(0, STEPS, gstep, pen0 + (zi,) * NCHUNK)
        inv = carry[NCHUNK:]
        for cc in range(NCHUNK):
            idx_v[pl.ds(cc * LANES, LANES)] = inv[cc] * BATCH + b
        pltpu.async_copy(preds_hbm.at[idx_v], rows_v, gsem).wait()
        pltpu.sync_copy(rows_v, out_hbm.at[pl.ds(b * NUM_NODES, NUM_NODES)])


@functools.cache
def _sc_select():
    return pl.kernel(
        _sc_select_body,
        out_type=jax.ShapeDtypeStruct((BATCH * NUM_NODES, NODE_DIM),
                                      jnp.float32),
        mesh=plsc.VectorSubcoreMesh(core_axis_name="c", subcore_axis_name="s"),
        scratch_types=[
            pltpu.VMEM((BLK,), jnp.float32),
            pltpu.VMEM((BLK,), jnp.float32),
            pltpu.VMEM((NUM_NODES,), jnp.int32),
            pltpu.VMEM((NUM_NODES, NODE_DIM), jnp.float32),
            pltpu.VMEM((LANES,), jnp.float32),
            pltpu.VMEM((LANES,), jnp.int32),
            pltpu.SemaphoreType.DMA,
            pltpu.SemaphoreType.DMA,
            pltpu.SemaphoreType.DMA,
        ],
        compiler_params=pltpu.CompilerParams(needs_layout_passes=False),
    )


def kernel(emb, node_emb_encoded, teacher_forcing, W_in1, b_in1, W_in2, b_in2,
           W_ih0, W_hh0, b_ih0, b_hh0, W_ih1, W_hh1, b_ih1, b_hh1,
           W_out, b_out):
    del teacher_forcing
    a_in1 = W_in1.T
    a_in2 = W_in2.T
    a_cat0 = jnp.concatenate([W_ih0.T, W_hh0.T], axis=0)
    a_cat1 = jnp.concatenate([W_ih1.T, W_hh1.T], axis=0)
    bias0 = (b_ih0 + b_hh0)[None, :]
    bias1 = (b_ih1 + b_hh1)[None, :]
    a_out = W_out.T

    preds, d = pl.pallas_call(
        _decode_body,
        out_shape=[
            jax.ShapeDtypeStruct((STEPS, BATCH, NODE_DIM), jnp.float32),
            jax.ShapeDtypeStruct((BATCH, STEPS + 1, NUM_NODES), jnp.float32),
        ],
    )(emb, a_in1, b_in1[None, :], a_in2, b_in2[None, :],
      a_cat0, bias0, a_cat1, bias1, a_out, b_out[None, :], node_emb_encoded)

    out_flat = _sc_select()(d.reshape(-1), preds.reshape(-1, NODE_DIM))
    return out_flat.reshape(BATCH, NUM_NODES, NODE_DIM)


# speedup vs baseline: 11.1087x; 1.0172x over previous
import functools

import jax
import jax.numpy as jnp
from jax import lax
from jax.experimental import pallas as pl
from jax.experimental.pallas import tpu as pltpu
from jax.experimental.pallas import tpu_sc as plsc

EMB_DIM = 256
NODE_DIM = 128
HIDDEN = 512
NUM_NODES = 64
BATCH = 128
STEPS = NUM_NODES

SC_CORES = 2
SC_SUBCORES = 16
SC_WORKERS = SC_CORES * SC_SUBCORES
SAMPLES_PER_WORKER = BATCH // SC_WORKERS
LANES = 16
NCHUNK = NUM_NODES // LANES


def _mm(a, b):
    return jax.lax.dot_general(
        a, b, (((1,), (0,)), ((), ())), preferred_element_type=jnp.float32
    )


def _decode_body(emb_ref, a_in1_ref, b_in1_ref, a_in2_ref, b_in2_ref,
                 a_cat0_ref, bias0_ref, a_cat1_ref, bias1_ref,
                 a_out_ref, b_out_ref, enc_ref, preds_ref, d_ref):
    h = jax.nn.relu(_mm(emb_ref[...], a_in1_ref[...]) + b_in1_ref[...])
    hx = _mm(h, a_in2_ref[...]) + b_in2_ref[...]
    h0 = hx[:, :HIDDEN]
    h1 = hx[:, HIDDEN:]
    c0 = jnp.zeros((BATCH, HIDDEN), jnp.float32)
    c1 = jnp.zeros((BATCH, HIDDEN), jnp.float32)
    x = jnp.zeros((BATCH, NODE_DIM), jnp.float32)

    a_cat0 = a_cat0_ref[...]
    bias0 = bias0_ref[...]
    a_cat1 = a_cat1_ref[...]
    bias1 = bias1_ref[...]
    a_out = a_out_ref[...]
    b_out = b_out_ref[...]

    def cell(xh, c, a_cat, bias):
        g = _mm(xh, a_cat) + bias
        i = jax.nn.sigmoid(g[:, 0 * HIDDEN:1 * HIDDEN])
        f = jax.nn.sigmoid(g[:, 1 * HIDDEN:2 * HIDDEN])
        gg = jnp.tanh(g[:, 2 * HIDDEN:3 * HIDDEN])
        o = jax.nn.sigmoid(g[:, 3 * HIDDEN:4 * HIDDEN])
        c_new = f * c + i * gg
        h_new = o * jnp.tanh(c_new)
        return h_new, c_new

    def step(t, carry):
        x, h0, c0, h1, c1 = carry
        h0, c0 = cell(jnp.concatenate([x, h0], axis=1), c0, a_cat0, bias0)
        h1, c1 = cell(jnp.concatenate([h0, h1], axis=1), c1, a_cat1, bias1)
        x = _mm(h1, a_out) + b_out
        preds_ref[t] = x
        return (x, h0, c0, h1, c1)

    def step4(u, carry):
        for k in range(4):
            carry = step(4 * u + k, carry)
        return carry

    lax.fori_loop(0, STEPS // 4, step4, (x, h0, c0, h1, c1))

    p_all = preds_ref[...] * -2.0
    for j in range(BATCH):
        p = p_all[:, j, :]
        e = enc_ref[j]
        d_ref[j, 1:] = jax.lax.dot_general(
            p, e, (((1,), (1,)), ((), ())),
            preferred_element_type=jnp.float32)
        d_ref[j, 0] = jnp.sum(e * e, axis=1)


BLK = (STEPS + 1) * NUM_NODES


def _sc_select_body(d_hbm, preds_hbm, out_hbm, d_v0, d_v1, idx_v,
                    rows_v, red_f, red_i, sem0, sem1, gsem):
    wid = lax.axis_index("s") * SC_CORES + lax.axis_index("c")
    iota = lax.iota(jnp.int32, LANES)
    big = jnp.float32(1e30)

    def lane_min_f(v):
        for sh in (8, 4, 2, 1):
            red_f[...] = v
            v = jnp.minimum(v, plsc.load_gather(red_f, [iota ^ sh]))
        return v

    def lane_min_i(v):
        for sh in (8, 4, 2, 1):
            red_i[...] = v
            v = jnp.minimum(v, plsc.load_gather(red_i, [iota ^ sh]))
        return v

    b0 = wid * SAMPLES_PER_WORKER
    bufs = (d_v0, d_v1)
    sems = (sem0, sem1)
    copies = {0: pltpu.async_copy(d_hbm.at[pl.ds(b0 * BLK, BLK)], d_v0, sem0)}
    for j in range(SAMPLES_PER_WORKER):
        b = b0 + j
        copies[j].wait()
        if j + 1 < SAMPLES_PER_WORKER:
            copies[j + 1] = pltpu.async_copy(
                d_hbm.at[pl.ds((b + 1) * BLK, BLK)],
                bufs[(j + 1) % 2], sems[(j + 1) % 2])
        d_v = bufs[j % 2]

        def gstep(t, carry):
            pen = carry[:NCHUNK]
            inv = carry[NCHUNK:]
            m = [d_v[pl.ds((t + 1) * NUM_NODES + cc * LANES, LANES)] + pen[cc]
                 for cc in range(NCHUNK)]
            mm = jnp.minimum(jnp.minimum(m[0], m[1]),
                             jnp.minimum(m[2], m[3]))
            gmin = lane_min_f(mm)
            cand = [jnp.where(m[cc] == gmin, iota + cc * LANES,
                              jnp.int32(NUM_NODES))
                    for cc in range(NCHUNK)]
            ci = jnp.minimum(jnp.minimum(cand[0], cand[1]),
                             jnp.minimum(cand[2], cand[3]))
            idx = lane_min_i(ci)
            hit = [iota + cc * LANES == idx for cc in range(NCHUNK)]
            pen = [jnp.where(hit[cc], big, pen[cc]) for cc in range(NCHUNK)]
            inv = [jnp.where(hit[cc], t, inv[cc]) for cc in range(NCHUNK)]
            return tuple(pen) + tuple(inv)

        pen0 = tuple(d_v[pl.ds(cc * LANES, LANES)] for cc in range(NCHUNK))
        zi = jnp.zeros((LANES,), jnp.int32)
        carry = lax.fori_loop(0, STEPS, gstep, pen0 + (zi,) * NCHUNK)
        inv = carry[NCHUNK:]
        for cc in range(NCHUNK):
            idx_v[pl.ds(cc * LANES, LANES)] = inv[cc] * BATCH + b
        pltpu.async_copy(preds_hbm.at[idx_v], rows_v, gsem).wait()
        pltpu.sync_copy(rows_v, out_hbm.at[pl.ds(b * NUM_NODES, NUM_NODES)])


@functools.cache
def _sc_select():
    return pl.kernel(
        _sc_select_body,
        out_type=jax.ShapeDtypeStruct((BATCH * NUM_NODES, NODE_DIM),
                                      jnp.float32),
        mesh=plsc.VectorSubcoreMesh(core_axis_name="c", subcore_axis_name="s"),
        scratch_types=[
            pltpu.VMEM((BLK,), jnp.float32),
            pltpu.VMEM((BLK,), jnp.float32),
            pltpu.VMEM((NUM_NODES,), jnp.int32),
            pltpu.VMEM((NUM_NODES, NODE_DIM), jnp.float32),
            pltpu.VMEM((LANES,), jnp.float32),
            pltpu.VMEM((LANES,), jnp.int32),
            pltpu.SemaphoreType.DMA,
            pltpu.SemaphoreType.DMA,
            pltpu.SemaphoreType.DMA,
        ],
        compiler_params=pltpu.CompilerParams(needs_layout_passes=False),
    )


def kernel(emb, node_emb_encoded, teacher_forcing, W_in1, b_in1, W_in2, b_in2,
           W_ih0, W_hh0, b_ih0, b_hh0, W_ih1, W_hh1, b_ih1, b_hh1,
           W_out, b_out):
    del teacher_forcing
    a_in1 = W_in1.T
    a_in2 = W_in2.T
    a_cat0 = jnp.concatenate([W_ih0.T, W_hh0.T], axis=0)
    a_cat1 = jnp.concatenate([W_ih1.T, W_hh1.T], axis=0)
    bias0 = (b_ih0 + b_hh0)[None, :]
    bias1 = (b_ih1 + b_hh1)[None, :]
    a_out = W_out.T

    preds, d = pl.pallas_call(
        _decode_body,
        out_shape=[
            jax.ShapeDtypeStruct((STEPS, BATCH, NODE_DIM), jnp.float32),
            jax.ShapeDtypeStruct((BATCH, STEPS + 1, NUM_NODES), jnp.float32),
        ],
    )(emb, a_in1, b_in1[None, :], a_in2, b_in2[None, :],
      a_cat0, bias0, a_cat1, bias1, a_out, b_out[None, :], node_emb_encoded)

    out_flat = _sc_select()(d.reshape(-1), preds.reshape(-1, NODE_DIM))
    return out_flat.reshape(BATCH, NUM_NODES, NODE_DIM)
